# Initial kernel scaffold; baseline (speedup 1.0000x reference)
import jax
import jax.numpy as jnp
from jax.experimental import pallas as pl


def kernel(x, edge_index, edge_attr, batch, y, atom_emb, bond_emb, eps, W1, b1, g1, bt1, W2, b2, Wp, bp, beta):
    raise NotImplementedError("write your pallas kernel here")


# speedup vs baseline: 3.8441x; 3.8441x over previous
import functools

import jax
import jax.numpy as jnp
from jax import lax
from jax.experimental import pallas as pl
from jax.experimental.pallas import tpu as pltpu
from jax.experimental.pallas import tpu_sc as plsc

H = 128
NLAYERS = 3
NC, NS, LANES = 2, 16, 16
NW = NC * NS
N = 10000
NPAD = 10240
NG = 64
E = 320000
EC = 128
ECHUNKS = 79
EPT = EC * ECHUNKS
EPAD = EPT * NW
NPT = NPAD // NW
AC = 64
ROWS_PER_TILE = NPAD // NS
BN_INV = 1.0 / (1.0 + 1e-5) ** 0.5
NBOND = 1000


def _sc_mesh():
    return plsc.VectorSubcoreMesh(core_axis_name="c", subcore_axis_name="s")


def _prep_body(bond_ref, eat_ref, ecomb_ref, code_ref):
    b0 = bond_ref[0]
    b1 = bond_ref[1]
    b2 = bond_ref[2]
    t01 = (b0[:, None, :] + b1[None, :, :]).reshape(100, H)
    ecomb_ref[...] = (t01[:, None, :] + b2[None, :, :]).reshape(NBOND, H)
    code_ref[...] = eat_ref[0] * 100 + eat_ref[1] * 10 + eat_ref[2]


def _prep_call(bond_emb, eat):
    return pl.pallas_call(
        _prep_body,
        out_shape=(
            jax.ShapeDtypeStruct((NBOND, H), jnp.float32),
            jax.ShapeDtypeStruct(eat.shape[1:], jnp.int32),
        ),
    )(bond_emb, eat)


def _atom_body(xt_hbm, aemb_hbm, h0_hbm, idx_v, gbuf, obuf, sem):
    c = lax.axis_index("c")
    s = lax.axis_index("s")
    wid = c * NS + s
    base = wid * NPT

    def chunk(j, _):
        nbase = base + j * AC
        for i in range(9):
            pltpu.sync_copy(xt_hbm.at[pl.ds(i * NPAD + nbase, AC)],
                            idx_v.at[i])
        def fix(k, _):
            for i in range(9):
                sl = pl.ds(k * LANES, LANES)
                idx_v[i, sl] = idx_v[i, sl] + i * 120
            return 0
        lax.fori_loop(0, AC // LANES, fix, 0)
        cps = [pltpu.async_copy(aemb_hbm.at[idx_v.at[i]], gbuf.at[i], sem)
               for i in range(9)]
        for cp in cps:
            cp.wait()

        def row(a, _):
            for r in range(H // LANES):
                sl = pl.ds(r * LANES, LANES)
                acc = gbuf[0, a, sl]
                for i in range(1, 9):
                    acc = acc + gbuf[i, a, sl]
                obuf[a, sl] = acc
            return 0
        lax.fori_loop(0, AC, row, 0)
        pltpu.sync_copy(obuf, h0_hbm.at[pl.ds(nbase, AC)])
        return 0

    lax.fori_loop(0, NPT // AC, chunk, 0)


def _atom_call(xt_flat, aemb_flat):
    k = functools.partial(
        pl.kernel,
        out_type=jax.ShapeDtypeStruct((NPAD, H), jnp.float32),
        mesh=_sc_mesh(),
        scratch_types=[
            pltpu.VMEM((9, AC), jnp.int32),
            pltpu.VMEM((9, AC, H), jnp.float32),
            pltpu.VMEM((AC, H), jnp.float32),
            pltpu.SemaphoreType.DMA,
        ],
    )(_atom_body)
    return k(xt_flat, aemb_flat)


def _msg_body(h_hbm, src_hbm, dst_hbm, code_hbm, ecomb_hbm, aggp_hbm,
              src_v, dst_v, code_v, hbuf, ebuf, ecomb_sp, agg_sp,
              sem1, sem2):
    c = lax.axis_index("c")
    s = lax.axis_index("s")
    wid = c * NS + s

    def zrow(e, _):
        for r in range(H // LANES):
            hbuf[e, pl.ds(r * LANES, LANES)] = jnp.zeros((LANES,), jnp.float32)
        return 0
    lax.fori_loop(0, EC, zrow, 0)
    for i in range(ROWS_PER_TILE // EC):
        pltpu.sync_copy(hbuf, agg_sp.at[pl.ds(s * ROWS_PER_TILE + i * EC, EC)])

    @pl.when(s == 0)
    def _():
        pltpu.sync_copy(ecomb_hbm, ecomb_sp)

    plsc.subcore_barrier()

    ebase = wid * EPT

    def chunk(j, _):
        off = ebase + j * EC
        pltpu.sync_copy(src_hbm.at[pl.ds(off, EC)], src_v)
        pltpu.sync_copy(code_hbm.at[pl.ds(off, EC)], code_v)
        pltpu.sync_copy(dst_hbm.at[pl.ds(off, EC)], dst_v)
        cp1 = pltpu.async_copy(h_hbm.at[src_v], hbuf, sem1)
        cp2 = pltpu.async_copy(ecomb_sp.at[code_v], ebuf, sem2)
        cp1.wait()
        cp2.wait()

        def row(e, _):
            for r in range(H // LANES):
                sl = pl.ds(r * LANES, LANES)
                hbuf[e, sl] = jnp.maximum(hbuf[e, sl] + ebuf[e, sl], 0.0)
            return 0
        lax.fori_loop(0, EC, row, 0)
        pltpu.sync_copy(hbuf, agg_sp.at[dst_v], add=True)
        return 0

    lax.fori_loop(0, ECHUNKS, chunk, 0)

    plsc.subcore_barrier()
    rbase = s * ROWS_PER_TILE
    pltpu.sync_copy(agg_sp.at[pl.ds(rbase, ROWS_PER_TILE)],
                    aggp_hbm.at[pl.ds(c * NPAD + rbase, ROWS_PER_TILE)])


def _msg_call(h, srcp, dstp, codep, ecomb):
    k = functools.partial(
        pl.kernel,
        out_type=jax.ShapeDtypeStruct((NC * NPAD, H), jnp.float32),
        mesh=_sc_mesh(),
        scratch_types=[
            pltpu.VMEM((EC,), jnp.int32),
            pltpu.VMEM((EC,), jnp.int32),
            pltpu.VMEM((EC,), jnp.int32),
            pltpu.VMEM((EC, H), jnp.float32),
            pltpu.VMEM((EC, H), jnp.float32),
            pltpu.VMEM_SHARED((NBOND, H), jnp.float32),
            pltpu.VMEM_SHARED((NPAD, H), jnp.float32),
            pltpu.SemaphoreType.DMA,
            pltpu.SemaphoreType.DMA,
        ],
    )(_msg_body)
    return k(h, srcp, dstp, codep, ecomb)


def _mlp_body(eps_ref, h_ref, a0_ref, a1_ref, w1_ref, b1_ref, g1_ref,
              bt1_ref, w2_ref, b2_ref, out_ref):
    scal = 1.0 + eps_ref[0, 0]
    z = scal * h_ref[...] + a0_ref[...] + a1_ref[...]
    z = jnp.dot(z, w1_ref[...], preferred_element_type=jnp.float32,
                precision=lax.Precision.HIGHEST)
    z = (z + b1_ref[...]) * (g1_ref[...] * BN_INV) + bt1_ref[...]
    z = jnp.maximum(z, 0.0)
    out_ref[...] = jnp.dot(z, w2_ref[...], preferred_element_type=jnp.float32,
                           precision=lax.Precision.HIGHEST) + b2_ref[...]


def _mlp_call(h, a0, a1, eps_l, w1, b1, g1, bt1, w2, b2):
    blk = 1024
    grid = NPAD // blk
    row_spec = pl.BlockSpec((blk, H), lambda i: (i, 0))
    full = lambda shape: pl.BlockSpec(shape, lambda i: (0, 0))
    return pl.pallas_call(
        _mlp_body,
        grid=(grid,),
        in_specs=[
            pl.BlockSpec(memory_space=pltpu.SMEM),
            row_spec, row_spec, row_spec,
            full((H, 2 * H)), full((1, 2 * H)), full((1, 2 * H)),
            full((1, 2 * H)), full((2 * H, H)), full((1, H)),
        ],
        out_specs=row_spec,
        out_shape=jax.ShapeDtypeStruct((NPAD, H), jnp.float32),
    )(eps_l, h, a0, a1, w1, b1, g1, bt1, w2, b2)


def _pool_body(consts_ref, batch_ref, h_ref, y_ref, wp_ref, out_ref,
               sums, counts):
    i = pl.program_id(0)
    nsteps = pl.num_programs(0)

    @pl.when(i == 0)
    def _():
        sums[...] = jnp.zeros_like(sums)
        counts[...] = jnp.zeros_like(counts)

    b = batch_ref[0, 0, :]
    oh = (b[:, None] == lax.broadcasted_iota(jnp.int32, (b.shape[0], NG), 1)
          ).astype(jnp.float32)
    dims = (((0,), (0,)), ((), ()))
    sums[...] += lax.dot_general(oh, h_ref[...], dims,
                                 precision=lax.Precision.HIGHEST,
                                 preferred_element_type=jnp.float32)
    counts[...] += lax.dot_general(oh, jnp.ones_like(h_ref[...]), dims,
                                   precision=lax.Precision.HIGHEST,
                                   preferred_element_type=jnp.float32)

    @pl.when(i == nsteps - 1)
    def _():
        bp = consts_ref[0, 0]
        beta = consts_ref[0, 1]
        hg = sums[...] / jnp.maximum(counts[...], 1.0)
        t = jnp.dot(hg, wp_ref[...], preferred_element_type=jnp.float32,
                    precision=lax.Precision.HIGHEST) + bp
        gp = 1.0 / (1.0 + jnp.exp(-t))
        y2 = y_ref[:, 2:3]
        za = gp * beta
        zb = y2 * beta
        m = jnp.maximum(za, zb)
        ea = jnp.exp(za - m)
        eb = jnp.exp(zb - m)
        out_ref[...] = (gp * ea + y2 * eb) / (ea + eb)


def _pool_call(h, batch2d, y, wp, consts):
    blk = 1024
    grid = NPAD // blk
    return pl.pallas_call(
        _pool_body,
        grid=(grid,),
        in_specs=[
            pl.BlockSpec(memory_space=pltpu.SMEM),
            pl.BlockSpec((1, 1, blk), lambda i: (i, 0, 0)),
            pl.BlockSpec((blk, H), lambda i: (i, 0)),
            pl.BlockSpec((NG, 3), lambda i: (0, 0)),
            pl.BlockSpec((H, 1), lambda i: (0, 0)),
        ],
        out_specs=pl.BlockSpec((NG, 1), lambda i: (0, 0)),
        out_shape=jax.ShapeDtypeStruct((NG, 1), jnp.float32),
        scratch_shapes=[
            pltpu.VMEM((NG, H), jnp.float32),
            pltpu.VMEM((NG, H), jnp.float32),
        ],
    )(consts, batch2d, h, y, wp)


def kernel(x, edge_index, edge_attr, batch, y, atom_emb, bond_emb, eps,
           W1, b1, g1, bt1, W2, b2, Wp, bp, beta):
    x = x.astype(jnp.int32)
    edge_attr = edge_attr.astype(jnp.int32)
    edge_index = edge_index.astype(jnp.int32)
    batch = batch.astype(jnp.int32)

    eat = edge_attr.T.reshape(3, E // H, H)
    ecomb, code2d = _prep_call(bond_emb, eat)
    code = code2d.reshape(E)

    pad = EPAD - E
    srcp = jnp.concatenate([edge_index[0], jnp.zeros((pad,), jnp.int32)])
    dstp = jnp.concatenate([edge_index[1],
                            jnp.full((pad,), NPAD - 1, jnp.int32)])
    codep = jnp.concatenate([code, jnp.zeros((pad,), jnp.int32)])

    xt = jnp.pad(x.T, ((0, 0), (0, NPAD - N))).reshape(9 * NPAD)
    aemb_flat = atom_emb.reshape(9 * 120, H)
    h = _atom_call(xt, aemb_flat)

    for l in range(NLAYERS):
        aggp = _msg_call(h, srcp, dstp, codep, ecomb)
        h = _mlp_call(h, aggp[:NPAD], aggp[NPAD:], eps[l].reshape(1, 1),
                      W1[l], b1[l].reshape(1, 2 * H), g1[l].reshape(1, 2 * H),
                      bt1[l].reshape(1, 2 * H), W2[l], b2[l].reshape(1, H))

    batchp = jnp.concatenate([batch, jnp.full((NPAD - N,), NG, jnp.int32)])
    batch2d = batchp.reshape(NPAD // 1024, 1, 1024)
    consts = jnp.stack([bp[0], beta[0]]).reshape(1, 2)
    return _pool_call(h, batch2d, y, Wp, consts)
